# baseline (speedup 1.0000x reference)
import functools

import jax
import jax.numpy as jnp
from jax import lax
from jax.experimental import pallas as pl
from jax.experimental.pallas import tpu as pltpu
from jax.experimental.pallas import tpu_sc as plsc

_VOCAB = 100000
_DIM = 300
_DIMP = 384
_B = 4096
_L = 20
_ROWS = _B * _L
_NC, _NS = 2, 16
_NW = _NC * _NS
_RPW = _ROWS // _NW
_CHUNK = 128
_NCH = _RPW // _CHUNK

_mesh = plsc.VectorSubcoreMesh(core_axis_name="c", subcore_axis_name="s")


@functools.partial(
    pl.kernel,
    mesh=_mesh,
    out_type=jax.ShapeDtypeStruct((_ROWS, _DIMP), jnp.float32),
    scratch_types=[
        pltpu.VMEM((_RPW,), jnp.int32),
        pltpu.VMEM((_CHUNK, _DIMP), jnp.float32),
        pltpu.VMEM((_CHUNK, _DIMP), jnp.float32),
        pltpu.SemaphoreType.DMA,
        pltpu.SemaphoreType.DMA,
    ],
)
def _sc_gather(idx_hbm, table_hbm, out_hbm, idx_v, buf0, buf1, sem0, sem1):
    wid = lax.axis_index("s") * _NC + lax.axis_index("c")
    base = wid * _RPW
    pltpu.sync_copy(idx_hbm.at[pl.ds(base, _RPW)], idx_v)
    bufs = (buf0, buf1)
    sems = (sem0, sem1)
    copies = [None] * _NCH
    copies[0] = pltpu.async_copy(
        table_hbm.at[idx_v.at[pl.ds(0, _CHUNK)]], bufs[0], sems[0])
    for c in range(_NCH):
        if c + 1 < _NCH:
            copies[c + 1] = pltpu.async_copy(
                table_hbm.at[idx_v.at[pl.ds((c + 1) * _CHUNK, _CHUNK)]],
                bufs[(c + 1) % 2], sems[(c + 1) % 2])
        copies[c].wait()
        pltpu.sync_copy(bufs[c % 2],
                        out_hbm.at[pl.ds(base + c * _CHUNK, _CHUNK)])


_BPC = 512
_NPC = _B // _BPC


def _pool_body(e_ref, x_ref):
    e = e_ref[...]
    ss = jnp.sum(e * e, axis=2, keepdims=True)
    norm = jnp.sqrt(ss)
    scale = jnp.minimum(1.0, 1.0 / jnp.maximum(norm, 1e-7))
    x = jnp.sum(e * scale, axis=0) * (1.0 / _L)
    x_ref[...] = x[:, :_DIM].astype(jnp.bfloat16)


def _pool(emb3):
    return pl.pallas_call(
        _pool_body,
        grid=(_NPC,),
        in_specs=[pl.BlockSpec((_L, _BPC, _DIMP), lambda i: (0, i, 0))],
        out_specs=pl.BlockSpec((_BPC, _DIM), lambda i: (i, 0)),
        out_shape=jax.ShapeDtypeStruct((_B, _DIM), jnp.bfloat16),
    )(emb3)


_BN = 6400
_NV = 16
_VALIGN = (_VOCAB // 128) * 128
_VTAIL = _VOCAB - _VALIGN
_BNL = _VALIGN - (_NV - 1) * _BN
_BM = 512
_NM = _B // _BM
_NBUF = 3
_NSTEPS = _NV * _NM


def _out_copy(s_ref, k, o_hbm, pj, pi, sem, wait):
    def _mk(width):
        return pltpu.make_async_copy(
            s_ref.at[k, :, pl.ds(0, width)],
            o_hbm.at[pl.ds(pi * _BM, _BM), pl.ds(pj * _BN, width)],
            sem.at[k])

    @pl.when(pj < _NV - 1)
    def _full():
        (_mk(_BN).wait() if wait else _mk(_BN).start())

    @pl.when(pj == _NV - 1)
    def _last():
        (_mk(_BNL).wait() if wait else _mk(_BNL).start())


def _mm_body(x_ref, w_ref, b_ref, o_hbm, s_ref, sem):
    j = pl.program_id(0)
    i = pl.program_id(1)
    step = j * _NM + i
    k = step % _NBUF

    @pl.when(step >= _NBUF)
    def _wait_prev():
        prev = step - _NBUF
        _out_copy(s_ref, k, o_hbm, prev // _NM, prev % _NM, sem, wait=True)

    acc = lax.dot_general(x_ref[...], w_ref[...], (((1,), (1,)), ((), ())),
                          preferred_element_type=jnp.float32)
    s_ref[k] = acc + b_ref[...]
    _out_copy(s_ref, k, o_hbm, j, i, sem, wait=False)

    @pl.when(step == _NSTEPS - 1)
    def _drain():
        for d in range(_NBUF):
            ps = _NSTEPS - _NBUF + d
            _out_copy(s_ref, ps % _NBUF, o_hbm, ps // _NM, ps % _NM,
                      sem, wait=True)


def _matmul(xbf, Wbf, bp2):
    return pl.pallas_call(
        _mm_body,
        grid=(_NV, _NM),
        in_specs=[
            pl.BlockSpec((_BM, _DIM), lambda j, i: (i, 0)),
            pl.BlockSpec((_BN, _DIM), lambda j, i: (j, 0)),
            pl.BlockSpec((1, _BN), lambda j, i: (0, j)),
        ],
        out_specs=pl.BlockSpec(memory_space=pl.ANY),
        out_shape=jax.ShapeDtypeStruct((_B, _VOCAB), jnp.float32),
        scratch_shapes=[
            pltpu.VMEM((_NBUF, _BM, _BN), jnp.float32),
            pltpu.SemaphoreType.DMA((_NBUF,)),
        ],
    )(xbf, Wbf, bp2)


def _tail_body(o_in_ref, x_ref, w_ref, b_ref, o_ref):
    acc = lax.dot_general(x_ref[...], w_ref[...], (((1,), (1,)), ((), ())),
                          preferred_element_type=jnp.float32)
    o_ref[...] = acc + b_ref[...]


def _tail(logits, xbf, wtp, btp):
    return pl.pallas_call(
        _tail_body,
        grid=(1,),
        in_specs=[
            pl.BlockSpec(memory_space=pl.ANY),
            pl.BlockSpec((_B, _DIM), lambda g: (0, 0)),
            pl.BlockSpec((128, _DIM), lambda g: (0, 0)),
            pl.BlockSpec((1, 128), lambda g: (0, 0)),
        ],
        out_specs=pl.BlockSpec((_B, 128), lambda g: (0, _VALIGN // 128)),
        out_shape=jax.ShapeDtypeStruct((_B, _VOCAB), jnp.float32),
        input_output_aliases={0: 0},
    )(logits, xbf, wtp, btp)


def kernel(inputs, table, W, b):
    idx = inputs.T.reshape(-1).astype(jnp.int32)
    table_p = jnp.pad(table, ((0, 0), (0, _DIMP - _DIM)))
    emb = _sc_gather(idx, table_p)
    xbf = _pool(emb.reshape(_L, _B, _DIMP))
    wbf = W.astype(jnp.bfloat16)
    bp2 = jnp.pad(b, (0, _NV * _BN - _VOCAB)).reshape(1, -1)
    wtp = jnp.pad(lax.slice(W, (_VALIGN, 0), (_VOCAB, _DIM)),
                  ((0, 128 - _VTAIL), (0, 0))).astype(jnp.bfloat16)
    btp = jnp.pad(lax.slice(b, (_VALIGN,), (_VOCAB,)),
                  (0, 128 - _VTAIL)).reshape(1, -1)
    logits = _matmul(xbf, wbf, bp2)
    return _tail(logits, xbf, wtp, btp)


# speedup vs baseline: 1.8859x; 1.8859x over previous
import functools

import jax
import jax.numpy as jnp
from jax import lax
from jax.experimental import pallas as pl
from jax.experimental.pallas import tpu as pltpu
from jax.experimental.pallas import tpu_sc as plsc

_VOCAB = 100000
_DIM = 300
_DIMP = 384
_B = 4096
_L = 20
_ROWS = _B * _L
_NC, _NS = 2, 16
_NW = _NC * _NS
_RPW = _ROWS // _NW
_CHUNK = 128
_NCH = _RPW // _CHUNK

_mesh = plsc.VectorSubcoreMesh(core_axis_name="c", subcore_axis_name="s")


@functools.partial(
    pl.kernel,
    mesh=_mesh,
    out_type=jax.ShapeDtypeStruct((_ROWS, _DIMP), jnp.float32),
    scratch_types=[
        pltpu.VMEM((_RPW,), jnp.int32),
        pltpu.VMEM((_CHUNK, _DIMP), jnp.float32),
        pltpu.VMEM((_CHUNK, _DIMP), jnp.float32),
        pltpu.SemaphoreType.DMA,
        pltpu.SemaphoreType.DMA,
    ],
)
def _sc_gather(idx_hbm, table_hbm, out_hbm, idx_v, buf0, buf1, sem0, sem1):
    wid = lax.axis_index("s") * _NC + lax.axis_index("c")
    base = wid * _RPW
    pltpu.sync_copy(idx_hbm.at[pl.ds(base, _RPW)], idx_v)
    bufs = (buf0, buf1)
    sems = (sem0, sem1)
    copies = [None] * _NCH
    copies[0] = pltpu.async_copy(
        table_hbm.at[idx_v.at[pl.ds(0, _CHUNK)]], bufs[0], sems[0])
    for c in range(_NCH):
        if c + 1 < _NCH:
            copies[c + 1] = pltpu.async_copy(
                table_hbm.at[idx_v.at[pl.ds((c + 1) * _CHUNK, _CHUNK)]],
                bufs[(c + 1) % 2], sems[(c + 1) % 2])
        copies[c].wait()
        pltpu.sync_copy(bufs[c % 2],
                        out_hbm.at[pl.ds(base + c * _CHUNK, _CHUNK)])


_BPC = 512
_NPC = _B // _BPC


def _pool_body(e_ref, x_ref):
    e = e_ref[...]
    ss = jnp.sum(e * e, axis=2, keepdims=True)
    norm = jnp.sqrt(ss)
    scale = jnp.minimum(1.0, 1.0 / jnp.maximum(norm, 1e-7))
    x = jnp.sum(e * scale, axis=0) * (1.0 / _L)
    x_ref[...] = x[:, :_DIM].astype(jnp.bfloat16)


def _pool(emb3):
    return pl.pallas_call(
        _pool_body,
        grid=(_NPC,),
        in_specs=[pl.BlockSpec((_L, _BPC, _DIMP), lambda i: (0, i, 0))],
        out_specs=pl.BlockSpec((_BPC, _DIM), lambda i: (i, 0)),
        out_shape=jax.ShapeDtypeStruct((_B, _DIM), jnp.bfloat16),
    )(emb3)


_BNV = 1024
_NVB = (_VOCAB + _BNV - 1) // _BNV


def _mmT_body(w_ref, xt_ref, b_ref, o_ref):
    acc = lax.dot_general(w_ref[...], xt_ref[...], (((1,), (0,)), ((), ())),
                          preferred_element_type=jnp.float32)
    o_ref[...] = acc + b_ref[...]


def _matmul_t(Wbf, xT, bcol):
    return pl.pallas_call(
        _mmT_body,
        grid=(_NVB,),
        in_specs=[
            pl.BlockSpec((_BNV, _DIM), lambda j: (j, 0)),
            pl.BlockSpec((_DIM, _B), lambda j: (0, 0)),
            pl.BlockSpec((_BNV, 1), lambda j: (j, 0)),
        ],
        out_specs=pl.BlockSpec((_BNV, _B), lambda j: (j, 0)),
        out_shape=jax.ShapeDtypeStruct((_VOCAB, _B), jnp.float32),
    )(Wbf, xT, bcol)


def kernel(inputs, table, W, b):
    idx = inputs.T.reshape(-1).astype(jnp.int32)
    table_p = jnp.pad(table, ((0, 0), (0, _DIMP - _DIM)))
    emb = _sc_gather(idx, table_p)
    xbf = _pool(emb.reshape(_L, _B, _DIMP))
    logits_t = _matmul_t(W.astype(jnp.bfloat16), xbf.T, b.reshape(-1, 1))
    return logits_t.T
